# baseline (speedup 1.0000x reference)
import functools

import jax
import jax.numpy as jnp
from jax import lax
from jax.experimental import pallas as pl
from jax.experimental.pallas import tpu as pltpu
from jax.experimental.pallas import tpu_sc as plsc

VOCAB_ROWS = 30000
DIM = 100

NUM_CORES = 2
NUM_SUBCORES = 16
LANES = 16
NUM_WORKERS = NUM_CORES * NUM_SUBCORES

CHUNK = 1024
NCHUNK = (VOCAB_ROWS + CHUNK - 1) // CHUNK
VOCAB_PAD = NCHUNK * CHUNK


def _tw_body(w_ref, b_ref, tt_ref, out_ref):
    acc = jax.lax.dot_general(
        w_ref[...], tt_ref[...],
        dimension_numbers=(((1,), (0,)), ((), ())),
        preferred_element_type=jnp.float32,
    )
    out_ref[...] = acc.reshape(CHUNK) + b_ref[0]


def _precompute_tw(table_t, W, b):
    return pl.pallas_call(
        _tw_body,
        grid=(NCHUNK,),
        in_specs=[
            pl.BlockSpec((1, DIM), lambda i: (0, 0)),
            pl.BlockSpec(memory_space=pltpu.SMEM),
            pl.BlockSpec((DIM, CHUNK), lambda i: (0, i)),
        ],
        out_specs=pl.BlockSpec((CHUNK,), lambda i: (i,)),
        out_shape=jax.ShapeDtypeStruct((VOCAB_PAD,), jnp.float32),
    )(W, b, table_t)


def _gather_kernel(batch, seq):
    cols_per_w = batch // NUM_WORKERS
    mesh = plsc.VectorSubcoreMesh(
        core_axis_name="c", subcore_axis_name="s",
        num_cores=NUM_CORES, num_subcores=NUM_SUBCORES)

    @functools.partial(
        pl.kernel,
        mesh=mesh,
        out_type=jax.ShapeDtypeStruct((seq, batch), jnp.float32),
        scratch_types=[
            pltpu.VMEM((VOCAB_PAD,), jnp.float32),
            pltpu.VMEM((seq, cols_per_w), jnp.int32),
            pltpu.VMEM((seq, cols_per_w), jnp.float32),
        ],
        compiler_params=pltpu.CompilerParams(
            needs_layout_passes=False, use_tc_tiling_on_sc=False),
    )
    def gather(tw_hbm, xt_hbm, out_hbm, tw_v, x_v, out_v):
        wid = lax.axis_index("s") * NUM_CORES + lax.axis_index("c")
        col0 = wid * cols_per_w
        pltpu.sync_copy(tw_hbm, tw_v)
        pltpu.sync_copy(xt_hbm.at[:, pl.ds(col0, cols_per_w)], x_v)

        @plsc.parallel_loop(0, seq, 1, unroll=4)
        def body(l):
            for j0 in range(0, cols_per_w, LANES):
                idx16 = x_v[l, pl.ds(j0, LANES)]
                out_v[l, pl.ds(j0, LANES)] = plsc.load_gather(tw_v, [idx16])

        pltpu.sync_copy(out_v, out_hbm.at[:, pl.ds(col0, cols_per_w)])

    return gather


def kernel(x, table, W, b):
    B, L = x.shape
    table_t = pltpu.with_memory_space_constraint(
        table.T, pltpu.MemorySpace.HBM)
    tw = _precompute_tw(table_t, W, b)
    out_t = _gather_kernel(B, L)(tw, x.T)
    return jnp.expand_dims(out_t.T, -1)


# speedup vs baseline: 1.1843x; 1.1843x over previous
import functools

import jax
import jax.numpy as jnp
from jax import lax
from jax.experimental import pallas as pl
from jax.experimental.pallas import tpu as pltpu
from jax.experimental.pallas import tpu_sc as plsc

VOCAB_ROWS = 30000
DIM = 100

NUM_CORES = 2
NUM_SUBCORES = 16
LANES = 16
NUM_WORKERS = NUM_CORES * NUM_SUBCORES

CHUNK = 2048
NCHUNK = (VOCAB_ROWS + CHUNK - 1) // CHUNK
VOCAB_PAD = NCHUNK * CHUNK


def _tw_body(w_ref, b_ref, tt_ref, out_ref):
    acc = jax.lax.dot_general(
        w_ref[...], tt_ref[...],
        dimension_numbers=(((1,), (0,)), ((), ())),
        preferred_element_type=jnp.float32,
    )
    out_ref[...] = acc.reshape(CHUNK) + b_ref[0]


def _precompute_tw(table_t, W, b):
    return pl.pallas_call(
        _tw_body,
        grid=(NCHUNK,),
        in_specs=[
            pl.BlockSpec((1, DIM), lambda i: (0, 0)),
            pl.BlockSpec(memory_space=pltpu.SMEM),
            pl.BlockSpec((DIM, CHUNK), lambda i: (0, i)),
        ],
        out_specs=pl.BlockSpec((CHUNK,), lambda i: (i,)),
        out_shape=jax.ShapeDtypeStruct((VOCAB_PAD,), jnp.float32),
    )(W, b, table_t)


def _gather_kernel(batch, seq):
    cols_per_w = batch // NUM_WORKERS
    mesh = plsc.VectorSubcoreMesh(
        core_axis_name="c", subcore_axis_name="s",
        num_cores=NUM_CORES, num_subcores=NUM_SUBCORES)

    @functools.partial(
        pl.kernel,
        mesh=mesh,
        out_type=jax.ShapeDtypeStruct((seq, batch), jnp.float32),
        scratch_types=[
            pltpu.VMEM((VOCAB_PAD,), jnp.float32),
            pltpu.VMEM((seq, cols_per_w), jnp.int32),
            pltpu.VMEM((seq, cols_per_w), jnp.float32),
        ],
        compiler_params=pltpu.CompilerParams(
            needs_layout_passes=False, use_tc_tiling_on_sc=False),
    )
    def gather(tw_hbm, xt_hbm, out_hbm, tw_v, x_v, out_v):
        wid = lax.axis_index("s") * NUM_CORES + lax.axis_index("c")
        col0 = wid * cols_per_w
        pltpu.sync_copy(tw_hbm, tw_v)
        pltpu.sync_copy(xt_hbm.at[:, pl.ds(col0, cols_per_w)], x_v)

        @plsc.parallel_loop(0, seq, 1, unroll=4)
        def body(l):
            for j0 in range(0, cols_per_w, LANES):
                idx16 = x_v[l, pl.ds(j0, LANES)]
                out_v[l, pl.ds(j0, LANES)] = plsc.load_gather(tw_v, [idx16])

        pltpu.sync_copy(out_v, out_hbm.at[:, pl.ds(col0, cols_per_w)])

    return gather


def kernel(x, table, W, b):
    B, L = x.shape
    table_t = pltpu.with_memory_space_constraint(
        table.T, pltpu.MemorySpace.HBM)
    tw = _precompute_tw(table_t, W, b)
    out_t = _gather_kernel(B, L)(tw, x.T)
    return jnp.expand_dims(out_t.T, -1)


# speedup vs baseline: 1.3042x; 1.1012x over previous
import functools

import jax
import jax.numpy as jnp
from jax import lax
from jax.experimental import pallas as pl
from jax.experimental.pallas import tpu as pltpu
from jax.experimental.pallas import tpu_sc as plsc

VOCAB_ROWS = 30000
DIM = 100

NUM_CORES = 2
NUM_SUBCORES = 16
LANES = 16
NUM_WORKERS = NUM_CORES * NUM_SUBCORES

CHUNK = 4096
NCHUNK = (VOCAB_ROWS + CHUNK - 1) // CHUNK
VOCAB_PAD = NCHUNK * CHUNK


def _tw_body(w_ref, b_ref, tt_ref, out_ref):
    acc = jax.lax.dot_general(
        w_ref[...], tt_ref[...],
        dimension_numbers=(((1,), (0,)), ((), ())),
        preferred_element_type=jnp.float32,
    )
    out_ref[...] = acc.reshape(CHUNK) + b_ref[0]


def _precompute_tw(table_t, W, b):
    return pl.pallas_call(
        _tw_body,
        grid=(NCHUNK,),
        in_specs=[
            pl.BlockSpec((1, DIM), lambda i: (0, 0)),
            pl.BlockSpec(memory_space=pltpu.SMEM),
            pl.BlockSpec((DIM, CHUNK), lambda i: (0, i)),
        ],
        out_specs=pl.BlockSpec((CHUNK,), lambda i: (i,)),
        out_shape=jax.ShapeDtypeStruct((VOCAB_PAD,), jnp.float32),
    )(W, b, table_t)


def _gather_kernel(batch, seq):
    cols_per_w = batch // NUM_WORKERS
    mesh = plsc.VectorSubcoreMesh(
        core_axis_name="c", subcore_axis_name="s",
        num_cores=NUM_CORES, num_subcores=NUM_SUBCORES)

    @functools.partial(
        pl.kernel,
        mesh=mesh,
        out_type=jax.ShapeDtypeStruct((seq, batch), jnp.float32),
        scratch_types=[
            pltpu.VMEM((VOCAB_PAD,), jnp.float32),
            pltpu.VMEM((seq, cols_per_w), jnp.int32),
            pltpu.VMEM((seq, cols_per_w), jnp.float32),
        ],
        compiler_params=pltpu.CompilerParams(
            needs_layout_passes=False, use_tc_tiling_on_sc=False),
    )
    def gather(tw_hbm, xt_hbm, out_hbm, tw_v, x_v, out_v):
        wid = lax.axis_index("s") * NUM_CORES + lax.axis_index("c")
        col0 = wid * cols_per_w
        pltpu.sync_copy(tw_hbm, tw_v)
        pltpu.sync_copy(xt_hbm.at[:, pl.ds(col0, cols_per_w)], x_v)

        @plsc.parallel_loop(0, seq, 1, unroll=2)
        def body(l):
            for j0 in range(0, cols_per_w, LANES):
                idx16 = x_v[l, pl.ds(j0, LANES)]
                out_v[l, pl.ds(j0, LANES)] = plsc.load_gather(tw_v, [idx16])

        pltpu.sync_copy(out_v, out_hbm.at[:, pl.ds(col0, cols_per_w)])

    return gather


def kernel(x, table, W, b):
    B, L = x.shape
    table_t = pltpu.with_memory_space_constraint(
        table.T, pltpu.MemorySpace.HBM)
    tw = _precompute_tw(table_t, W, b)
    out_t = _gather_kernel(B, L)(tw, x.T)
    return jnp.expand_dims(out_t.T, -1)


# speedup vs baseline: 1.3776x; 1.0563x over previous
import functools

import jax
import jax.numpy as jnp
from jax import lax
from jax.experimental import pallas as pl
from jax.experimental.pallas import tpu as pltpu
from jax.experimental.pallas import tpu_sc as plsc

VOCAB_ROWS = 30000
DIM = 100

NUM_CORES = 2
NUM_SUBCORES = 16
LANES = 16
NUM_WORKERS = NUM_CORES * NUM_SUBCORES

CHUNK = 8192
NCHUNK = (VOCAB_ROWS + CHUNK - 1) // CHUNK
VOCAB_PAD = NCHUNK * CHUNK


def _tw_body(w_ref, b_ref, tt_ref, out_ref):
    acc = jax.lax.dot_general(
        w_ref[...], tt_ref[...],
        dimension_numbers=(((1,), (0,)), ((), ())),
        preferred_element_type=jnp.float32,
    )
    out_ref[...] = acc.reshape(CHUNK) + b_ref[0]


def _precompute_tw(table_t, W, b):
    return pl.pallas_call(
        _tw_body,
        grid=(NCHUNK,),
        in_specs=[
            pl.BlockSpec((1, DIM), lambda i: (0, 0)),
            pl.BlockSpec(memory_space=pltpu.SMEM),
            pl.BlockSpec((DIM, CHUNK), lambda i: (0, i)),
        ],
        out_specs=pl.BlockSpec((CHUNK,), lambda i: (i,)),
        out_shape=jax.ShapeDtypeStruct((VOCAB_PAD,), jnp.float32),
    )(W, b, table_t)


def _gather_kernel(batch, seq):
    cols_per_w = batch // NUM_WORKERS
    mesh = plsc.VectorSubcoreMesh(
        core_axis_name="c", subcore_axis_name="s",
        num_cores=NUM_CORES, num_subcores=NUM_SUBCORES)

    @functools.partial(
        pl.kernel,
        mesh=mesh,
        out_type=jax.ShapeDtypeStruct((seq, batch), jnp.float32),
        scratch_types=[
            pltpu.VMEM((VOCAB_PAD,), jnp.float32),
            pltpu.VMEM((seq, cols_per_w), jnp.int32),
            pltpu.VMEM((seq, cols_per_w), jnp.float32),
        ],
        compiler_params=pltpu.CompilerParams(
            needs_layout_passes=False, use_tc_tiling_on_sc=False),
    )
    def gather(tw_hbm, xt_hbm, out_hbm, tw_v, x_v, out_v):
        wid = lax.axis_index("s") * NUM_CORES + lax.axis_index("c")
        col0 = wid * cols_per_w
        pltpu.sync_copy(tw_hbm, tw_v)
        pltpu.sync_copy(xt_hbm.at[:, pl.ds(col0, cols_per_w)], x_v)

        @plsc.parallel_loop(0, seq, 1, unroll=2)
        def body(l):
            for j0 in range(0, cols_per_w, LANES):
                idx16 = x_v[l, pl.ds(j0, LANES)]
                out_v[l, pl.ds(j0, LANES)] = plsc.load_gather(tw_v, [idx16])

        pltpu.sync_copy(out_v, out_hbm.at[:, pl.ds(col0, cols_per_w)])

    return gather


def kernel(x, table, W, b):
    B, L = x.shape
    table_t = pltpu.with_memory_space_constraint(
        table.T, pltpu.MemorySpace.HBM)
    tw = _precompute_tw(table_t, W, b)
    out_t = _gather_kernel(B, L)(tw, x.T)
    return jnp.expand_dims(out_t.T, -1)
